# baseline (speedup 1.0000x reference)
import functools

import jax
import jax.numpy as jnp
from jax import lax
from jax.experimental import pallas as pl
from jax.experimental.pallas import tpu as pltpu
from jax.experimental.pallas import tpu_sc as plsc

B = 16384
EMB = 16
NC, NS = 2, 16
NW = NC * NS
B_PER_W = B // NW


def _sc_gather(userId, movieId, user_table, movie_table):
    mesh = plsc.VectorSubcoreMesh(core_axis_name="c", subcore_axis_name="s")

    @functools.partial(
        pl.kernel,
        mesh=mesh,
        compiler_params=pltpu.CompilerParams(use_tc_tiling_on_sc=False),
        out_type=[
            jax.ShapeDtypeStruct((B, EMB), jnp.float32),
            jax.ShapeDtypeStruct((B, EMB), jnp.float32),
        ],
        scratch_types=[
            pltpu.VMEM((B_PER_W,), jnp.int32),
            pltpu.VMEM((B_PER_W, EMB), jnp.float32),
            pltpu.VMEM((B_PER_W,), jnp.int32),
            pltpu.VMEM((B_PER_W, EMB), jnp.float32),
            pltpu.SemaphoreType.DMA,
            pltpu.SemaphoreType.DMA,
        ],
    )
    def gather_kernel(uid_hbm, mid_hbm, ut_hbm, mt_hbm, ue_hbm, me_hbm,
                      uidx_v, urows_v, midx_v, mrows_v, usem, msem):
        wid = lax.axis_index("s") * NC + lax.axis_index("c")
        base = wid * B_PER_W
        pltpu.sync_copy(uid_hbm.at[pl.ds(base, B_PER_W)], uidx_v)
        pltpu.sync_copy(mid_hbm.at[pl.ds(base, B_PER_W)], midx_v)
        cu = pltpu.async_copy(ut_hbm.at[uidx_v], urows_v, usem)
        cm = pltpu.async_copy(mt_hbm.at[midx_v], mrows_v, msem)
        cu.wait()
        cm.wait()
        pltpu.sync_copy(urows_v, ue_hbm.at[pl.ds(base, B_PER_W)])
        pltpu.sync_copy(mrows_v, me_hbm.at[pl.ds(base, B_PER_W)])

    return gather_kernel(userId, movieId, user_table, movie_table)


def _mlp_body(u_ref, m_ref, w1u_ref, w1m_ref, b1_ref, w2t_ref, b2_ref,
              w3t_ref, b3_ref, o_ref):
    x1 = jnp.dot(u_ref[...], w1u_ref[...], preferred_element_type=jnp.float32)
    x1 += jnp.dot(m_ref[...], w1m_ref[...], preferred_element_type=jnp.float32)
    h1 = jnp.maximum(x1 + b1_ref[...], 0.0)
    h2 = jnp.maximum(
        jnp.dot(h1, w2t_ref[...], preferred_element_type=jnp.float32) + b2_ref[...],
        0.0,
    )
    o_ref[...] = (
        jnp.dot(h2, w3t_ref[...], preferred_element_type=jnp.float32) + b3_ref[...]
    )


def _tc_mlp(user_eb, movie_eb, W1u, W1m, b1, W2t, b2, W3t, b3):
    return pl.pallas_call(
        _mlp_body,
        out_shape=jax.ShapeDtypeStruct((B, 1), jnp.float32),
    )(user_eb, movie_eb, W1u, W1m, b1, W2t, b2, W3t, b3)


@jax.jit
def kernel(userId, movieId, user_table, movie_table, W1, b1, W2, b2, W3, b3):
    user_eb, movie_eb = _sc_gather(userId, movieId, user_table, movie_table)
    W1u = W1[:, :EMB].T
    W1m = W1[:, EMB:].T
    W2t = W2.T
    W3t = W3.T
    return _tc_mlp(user_eb, movie_eb, W1u, W1m, b1[None, :], W2t, b2[None, :],
                   W3t, b3[None, :])
